# baseline (speedup 1.0000x reference)
import functools

import jax
import jax.numpy as jnp
from jax import lax
from jax.experimental import pallas as pl
from jax.experimental.pallas import tpu as pltpu
from jax.experimental.pallas import tpu_sc as plsc

NUM_CORES = 2
NUM_SUBCORES = 16
NUM_WORKERS = NUM_CORES * NUM_SUBCORES

K = 4
ROW = 128


def _gather_sc(table128, ids2d, n_tok, n_per):
    tok_per_w = n_tok // NUM_WORKERS
    n_steps = tok_per_w // K
    assert n_steps % 2 == 0 and n_per <= 128

    mesh = plsc.VectorSubcoreMesh(
        core_axis_name="c", subcore_axis_name="s",
        num_cores=NUM_CORES, num_subcores=NUM_SUBCORES)

    @functools.partial(
        pl.kernel,
        mesh=mesh,
        out_type=jax.ShapeDtypeStruct((n_tok, n_per, ROW), jnp.float32),
        compiler_params=pltpu.CompilerParams(use_tc_tiling_on_sc=False),
        scratch_types=[
            pltpu.VMEM((tok_per_w, n_per), jnp.int32),
            pltpu.VMEM((2, K, n_per, ROW), jnp.float32),
            pltpu.SemaphoreType.DMA,
            pltpu.SemaphoreType.DMA,
            pltpu.SemaphoreType.DMA,
            pltpu.SemaphoreType.DMA,
        ],
    )
    def body(table_hbm, ids_hbm, out_hbm, idx_v, gbuf, g0, g1, w0, w1):
        wid = lax.axis_index("s") * NUM_CORES + lax.axis_index("c")
        tok_base = wid * tok_per_w
        gsem = (g0, g1)
        wsem = (w0, w1)
        pltpu.sync_copy(ids_hbm.at[pl.ds(tok_base, tok_per_w)], idx_v)

        def fire_gathers(s, b):
            for j in range(K):
                pltpu.async_copy(
                    table_hbm.at[idx_v.at[s * K + j]],
                    gbuf.at[b, j],
                    gsem[b])

        def wait_gathers(b):
            for j in range(K):
                pltpu.make_async_copy(
                    table_hbm.at[pl.ds(0, n_per)], gbuf.at[b, j],
                    gsem[b]).wait()

        def fire_write(s, b):
            pltpu.async_copy(
                gbuf.at[b],
                out_hbm.at[pl.ds(tok_base + s * K, K)],
                wsem[b])

        def wait_write(b):
            pltpu.make_async_copy(
                gbuf.at[b], out_hbm.at[pl.ds(tok_base, K)], wsem[b]).wait()

        fire_gathers(0, 0)

        @pl.loop(0, n_steps, step=2)
        def _steps(t):
            for b in range(2):
                s = t + b
                b2 = 1 - b

                @pl.when(s + 1 < n_steps)
                def _prefetch():
                    @pl.when(s >= 1)
                    def _drain():
                        wait_write(b2)
                    fire_gathers(s + 1, b2)

                wait_gathers(b)
                fire_write(s, b)

        wait_write(0)
        wait_write(1)

    return body(table128, ids2d)


def kernel(token_ids, Embedding_Matrix):
    n_tok, n_per = token_ids.shape
    d = Embedding_Matrix.shape[1]
    table128 = jnp.pad(Embedding_Matrix, ((0, 0), (0, ROW - d)))
    out128 = _gather_sc(table128, token_ids.astype(jnp.int32), n_tok, n_per)
    return out128[:, :, :d]


# speedup vs baseline: 1.0602x; 1.0602x over previous
import functools

import jax
import jax.numpy as jnp
from jax import lax
from jax.experimental import pallas as pl
from jax.experimental.pallas import tpu as pltpu
from jax.experimental.pallas import tpu_sc as plsc

NUM_CORES = 2
NUM_SUBCORES = 16
NUM_WORKERS = NUM_CORES * NUM_SUBCORES

K = 8


def _gather_sc(table, ids2d, n_tok, n_per, d):
    tok_per_w = n_tok // NUM_WORKERS
    n_steps = tok_per_w // K
    half_tok = n_tok // 2
    assert n_steps % 2 == 0 and n_per <= 128

    mesh = plsc.VectorSubcoreMesh(
        core_axis_name="c", subcore_axis_name="s",
        num_cores=NUM_CORES, num_subcores=NUM_SUBCORES)

    @functools.partial(
        pl.kernel,
        mesh=mesh,
        out_type=(
            jax.ShapeDtypeStruct((half_tok, n_per, d), jnp.float32),
            jax.ShapeDtypeStruct((half_tok, n_per, d), jnp.float32),
        ),
        compiler_params=pltpu.CompilerParams(use_tc_tiling_on_sc=False),
        scratch_types=[
            pltpu.VMEM((tok_per_w, n_per), jnp.int32),
            pltpu.VMEM((2, K, n_per, d), jnp.float32),
            pltpu.SemaphoreType.DMA,
            pltpu.SemaphoreType.DMA,
            pltpu.SemaphoreType.DMA,
            pltpu.SemaphoreType.DMA,
        ],
    )
    def body(table_hbm, ids_hbm, out0_hbm, out1_hbm, idx_v, rows_v,
             g0, g1, w0, w1):
        wid = lax.axis_index("s") * NUM_CORES + lax.axis_index("c")
        tok_base = wid * tok_per_w
        in_lo = tok_base < half_tok
        half_base = lax.select(in_lo, tok_base, tok_base - half_tok)
        gsem = (g0, g1)
        wsem = (w0, w1)
        pltpu.sync_copy(ids_hbm.at[pl.ds(tok_base, tok_per_w)], idx_v)

        def fire_gathers(s, b):
            for j in range(K):
                pltpu.async_copy(
                    table_hbm.at[idx_v.at[s * K + j]],
                    rows_v.at[b, j],
                    gsem[b])

        def wait_gathers(b):
            for j in range(K):
                pltpu.make_async_copy(
                    table_hbm.at[pl.ds(0, n_per)], rows_v.at[b, j],
                    gsem[b]).wait()

        def fire_write(s, b):
            @pl.when(in_lo)
            def _lo():
                pltpu.async_copy(
                    rows_v.at[b], out0_hbm.at[pl.ds(half_base + s * K, K)],
                    wsem[b])

            @pl.when(jnp.logical_not(in_lo))
            def _hi():
                pltpu.async_copy(
                    rows_v.at[b], out1_hbm.at[pl.ds(half_base + s * K, K)],
                    wsem[b])

        def wait_write(b):
            pltpu.make_async_copy(
                rows_v.at[b], out0_hbm.at[pl.ds(0, K)], wsem[b]).wait()

        fire_gathers(0, 0)

        @pl.loop(0, n_steps, step=2)
        def _steps(t):
            for b in range(2):
                s = t + b
                b2 = 1 - b

                @pl.when(s + 1 < n_steps)
                def _prefetch():
                    @pl.when(s >= 1)
                    def _drain():
                        wait_write(b2)
                    fire_gathers(s + 1, b2)

                wait_gathers(b)
                fire_write(s, b)

        wait_write(0)
        wait_write(1)

    return body(table, ids2d)


def kernel(token_ids, Embedding_Matrix):
    n_tok, n_per = token_ids.shape
    d = Embedding_Matrix.shape[1]
    o0, o1 = _gather_sc(Embedding_Matrix, token_ids.astype(jnp.int32),
                        n_tok, n_per, d)
    return jnp.concatenate([o0, o1], axis=0)


# speedup vs baseline: 1.1366x; 1.0720x over previous
import functools

import jax
import jax.numpy as jnp
from jax import lax
from jax.experimental import pallas as pl
from jax.experimental.pallas import tpu as pltpu
from jax.experimental.pallas import tpu_sc as plsc

NUM_CORES = 2
NUM_SUBCORES = 16
NUM_WORKERS = NUM_CORES * NUM_SUBCORES

K = 8


def _gather_sc(table, ids):
    n_tok, n_per = ids.shape
    d = table.shape[1]
    tok_per_w = n_tok // NUM_WORKERS
    n_steps = tok_per_w // K
    assert n_steps % 2 == 0 and n_per <= 128

    mesh = plsc.VectorSubcoreMesh(
        core_axis_name="c", subcore_axis_name="s",
        num_cores=NUM_CORES, num_subcores=NUM_SUBCORES)

    @functools.partial(
        pl.kernel,
        mesh=mesh,
        out_type=jax.ShapeDtypeStruct((n_tok, n_per, d), jnp.float32),
        compiler_params=pltpu.CompilerParams(use_tc_tiling_on_sc=False),
        scratch_types=[
            pltpu.VMEM((tok_per_w, n_per), jnp.int32),
            pltpu.VMEM((2, K, n_per, d), jnp.float32),
            pltpu.SemaphoreType.DMA,
            pltpu.SemaphoreType.DMA,
            pltpu.SemaphoreType.DMA,
            pltpu.SemaphoreType.DMA,
        ],
    )
    def body(table_hbm, ids_hbm, out_hbm, idx_v, rows_v, g0, g1, w0, w1):
        wid = lax.axis_index("s") * NUM_CORES + lax.axis_index("c")
        tok_base = wid * tok_per_w
        gsem = (g0, g1)
        wsem = (w0, w1)
        pltpu.sync_copy(ids_hbm.at[pl.ds(tok_base, tok_per_w)], idx_v)

        def fire_gathers(s, b):
            for j in range(K):
                pltpu.async_copy(
                    table_hbm.at[idx_v.at[s * K + j]],
                    rows_v.at[b, j],
                    gsem[b])

        def wait_gathers(b):
            for j in range(K):
                pltpu.make_async_copy(
                    table_hbm.at[pl.ds(0, n_per)], rows_v.at[b, j],
                    gsem[b]).wait()

        def fire_write(s, b):
            pltpu.async_copy(
                rows_v.at[b],
                out_hbm.at[pl.ds(tok_base + s * K, K)],
                wsem[b])

        def wait_write(b):
            pltpu.make_async_copy(
                rows_v.at[b], out_hbm.at[pl.ds(tok_base, K)], wsem[b]).wait()

        fire_gathers(0, 0)

        @pl.loop(0, n_steps, step=2)
        def _steps(t):
            for b in range(2):
                s = t + b
                b2 = 1 - b

                @pl.when(s + 1 < n_steps)
                def _prefetch():
                    @pl.when(s >= 1)
                    def _drain():
                        wait_write(b2)
                    fire_gathers(s + 1, b2)

                wait_gathers(b)
                fire_write(s, b)

        wait_write(0)
        wait_write(1)

    return body(table, ids)


def kernel(token_ids, Embedding_Matrix):
    return _gather_sc(Embedding_Matrix, token_ids.astype(jnp.int32))
